# baseline (speedup 1.0000x reference)
import math

import jax
import jax.numpy as jnp
from jax.experimental import pallas as pl
from jax.experimental.pallas import tpu as pltpu

NUM_HEADS = 16
KDIM = 64
VDIM = 128
NUM_BLOCKS = 64
TOPK = 16
EPS = 1e-08
INPUT_SIZE = 1024
HIDDEN_SIZE = 1024
QKD = NUM_HEADS * KDIM

_SCALE = 1.0 / (NUM_HEADS * math.sqrt(KDIM))

NG = 8
BT = 8

_NT = (((1,), (1,)), ((), ()))


def _q_kernel(h_ref, wg_ref, q_ref):
    for j in range(NG):
        q_ref[j] = jnp.dot(h_ref[:, j, :], wg_ref[j],
                           preferred_element_type=jnp.float32)


def _q_kernel2(h_ref, wg_ref, q_ref):
    for j in range(NG):
        q_ref[j] = jnp.dot(h_ref[:, j, :], wg_ref[j],
                           preferred_element_type=jnp.float32)


def _attn_kernel(x_ref, q_ref, wk_ref, wv_ref, out_ref, mask_ref, nn_ref):
    s = x_ref.shape[1]
    nb = NUM_BLOCKS
    xf = x_ref[...].reshape(BT * s, INPUT_SIZE)
    k = jnp.dot(xf, wk_ref[...], preferred_element_type=jnp.float32)
    v = jnp.dot(xf, wv_ref[...], preferred_element_type=jnp.float32)
    i_col = jax.lax.broadcasted_iota(jnp.int32, (nb, nb), 0)
    i_row = jax.lax.broadcasted_iota(jnp.int32, (nb, nb), 1)
    lt = i_row < i_col
    ones_nb1 = jnp.ones((nb, 1), dtype=jnp.float32)
    nn_cols = []
    msk_cols = []
    for j in range(BT):
        kj = k[j * s:(j + 1) * s, :]
        vj = v[j * s:(j + 1) * s, :]
        qj = q_ref[:, j, :]
        sc = jax.lax.dot_general(
            qj, kj, _NT, preferred_element_type=jnp.float32) * _SCALE
        m = jnp.max(sc, axis=0, keepdims=True)
        e = jnp.exp(sc - m)
        probs = e / jnp.sum(e, axis=0, keepdims=True)
        rowsum = jnp.sum(probs, axis=1, keepdims=True)
        nn = 1.0 - rowsum + probs[:, s - 1:s]
        nnr = jnp.broadcast_to(jnp.transpose(nn), (nb, nb))
        nnc = jnp.transpose(nnr)
        beats = ((nnr > nnc) | ((nnr == nnc) & lt)).astype(jnp.float32)
        rank = jnp.dot(beats, ones_nb1,
                       preferred_element_type=jnp.float32)
        msk_col = (rank < float(TOPK)).astype(jnp.float32)
        p2 = probs + EPS
        p2 = p2 / jnp.sum(p2, axis=1, keepdims=True)
        pv = jnp.dot(p2, vj, preferred_element_type=jnp.float32)
        out_ref[j] = pv * msk_col
        nn_cols.append(nn)
        msk_cols.append(msk_col)
    nn_all = jnp.concatenate(nn_cols, axis=1)
    msk_all = jnp.concatenate(msk_cols, axis=1)
    nn_ref[...] = jnp.transpose(nn_all)
    mask_ref[...] = jnp.transpose(msk_all)


@jax.jit
def kernel(x, h, W_key, W_value, W_group):
    B, S, _ = x.shape
    wv_eff = jnp.mean(W_value.reshape(INPUT_SIZE, NUM_HEADS, VDIM), axis=1)
    q = pl.pallas_call(
        _q_kernel2,
        grid=(NUM_BLOCKS // NG, 2),
        in_specs=[
            pl.BlockSpec((B, NG, HIDDEN_SIZE), lambda i, kk: (0, i, 0)),
            pl.BlockSpec((NG, HIDDEN_SIZE, QKD // 2), lambda i, kk: (i, 0, kk)),
        ],
        out_specs=pl.BlockSpec((NG, B, QKD // 2), lambda i, kk: (i, 0, kk)),
        out_shape=jax.ShapeDtypeStruct((NUM_BLOCKS, B, QKD), jnp.float32),
    )(h, W_group)

    out, mask, nn = pl.pallas_call(
        _attn_kernel,
        grid=(B // BT,),
        in_specs=[
            pl.BlockSpec((BT, S, INPUT_SIZE), lambda i: (i, 0, 0)),
            pl.BlockSpec((NUM_BLOCKS, BT, QKD), lambda i: (0, i, 0)),
            pl.BlockSpec((INPUT_SIZE, QKD), lambda i: (0, 0)),
            pl.BlockSpec((INPUT_SIZE, VDIM), lambda i: (0, 0)),
        ],
        out_specs=[
            pl.BlockSpec((BT, NUM_BLOCKS, VDIM), lambda i: (i, 0, 0)),
            pl.BlockSpec((BT, NUM_BLOCKS), lambda i: (i, 0)),
            pl.BlockSpec((BT, NUM_BLOCKS), lambda i: (i, 0)),
        ],
        out_shape=[
            jax.ShapeDtypeStruct((B, NUM_BLOCKS, VDIM), jnp.float32),
            jax.ShapeDtypeStruct((B, NUM_BLOCKS), jnp.float32),
            jax.ShapeDtypeStruct((B, NUM_BLOCKS), jnp.float32),
        ],
    )(x, q, W_key, wv_eff)

    return out, mask, jax.lax.stop_gradient(nn)


# speedup vs baseline: 1.3771x; 1.3771x over previous
import math

import jax
import jax.numpy as jnp
from jax.experimental import pallas as pl
from jax.experimental.pallas import tpu as pltpu

NUM_HEADS = 16
KDIM = 64
VDIM = 128
NUM_BLOCKS = 64
TOPK = 16
EPS = 1e-08
INPUT_SIZE = 1024
HIDDEN_SIZE = 1024
QKD = NUM_HEADS * KDIM

_SCALE = 1.0 / (NUM_HEADS * math.sqrt(KDIM))

NG = 8
BT = 8

_NT = (((1,), (1,)), ((), ()))


def _q_kernel(h_ref, wg_ref, q_ref):
    for j in range(NG):
        q_ref[j] = jnp.dot(h_ref[:, j, :], wg_ref[j],
                           preferred_element_type=jnp.float32)


def _q_kernel2(h_ref, wg_ref, q_ref):
    for j in range(NG):
        q_ref[:, j, :] = jnp.dot(h_ref[:, j, :], wg_ref[j],
                                 preferred_element_type=jnp.float32)


def _attn_kernel(x_ref, q_ref, wk_ref, wv_ref, out_ref, mask_ref, nn_ref):
    s = x_ref.shape[1]
    nb = NUM_BLOCKS
    xf = x_ref[...].reshape(BT * s, INPUT_SIZE)
    k = jnp.dot(xf, wk_ref[...], preferred_element_type=jnp.float32)
    v = jnp.dot(xf, wv_ref[...], preferred_element_type=jnp.float32)
    i_col = jax.lax.broadcasted_iota(jnp.int32, (nb, nb), 0)
    i_row = jax.lax.broadcasted_iota(jnp.int32, (nb, nb), 1)
    lt = i_row < i_col
    ones_nb1 = jnp.ones((nb, 1), dtype=jnp.float32)
    nn_cols = []
    msk_cols = []
    scs = []
    vjs = []
    for j in range(BT):
        kj = k[j * s:(j + 1) * s, :]
        vjs.append(v[j * s:(j + 1) * s, :])
        qj = q_ref[j]
        scs.append(jax.lax.dot_general(
            qj, kj, _NT, preferred_element_type=jnp.float32) * _SCALE)
    p2s = []
    for j in range(BT):
        sc = scs[j]
        m = jnp.max(sc, axis=0, keepdims=True)
        e = jnp.exp(sc - m)
        probs = e / jnp.sum(e, axis=0, keepdims=True)
        rowsum = jnp.sum(probs, axis=1, keepdims=True)
        nn = 1.0 - rowsum + probs[:, s - 1:s]
        nnr = jnp.broadcast_to(jnp.transpose(nn), (nb, nb))
        nnc = jnp.transpose(nnr)
        beats = ((nnr > nnc) | ((nnr == nnc) & lt)).astype(jnp.float32)
        rank = jnp.dot(beats, ones_nb1,
                       preferred_element_type=jnp.float32)
        msk_col = (rank < float(TOPK)).astype(jnp.float32)
        p2 = probs + EPS
        p2s.append(p2 / jnp.sum(p2, axis=1, keepdims=True))
        nn_cols.append(nn)
        msk_cols.append(msk_col)
    for j in range(BT):
        pv = jnp.dot(p2s[j], vjs[j],
                     preferred_element_type=jnp.float32)
        out_ref[j] = pv * msk_cols[j]
    nn_all = jnp.concatenate(nn_cols, axis=1)
    msk_all = jnp.concatenate(msk_cols, axis=1)
    nn_ref[...] = jnp.transpose(nn_all)
    mask_ref[...] = jnp.transpose(msk_all)


@jax.jit
def kernel(x, h, W_key, W_value, W_group):
    B, S, _ = x.shape
    wv_eff = jnp.mean(W_value.reshape(INPUT_SIZE, NUM_HEADS, VDIM), axis=1)
    q = pl.pallas_call(
        _q_kernel2,
        grid=(NUM_BLOCKS // NG, 2),
        in_specs=[
            pl.BlockSpec((B, NG, HIDDEN_SIZE), lambda i, kk: (0, i, 0)),
            pl.BlockSpec((NG, HIDDEN_SIZE, QKD // 2), lambda i, kk: (i, 0, kk)),
        ],
        out_specs=pl.BlockSpec((B, NG, QKD // 2), lambda i, kk: (0, i, kk)),
        out_shape=jax.ShapeDtypeStruct((B, NUM_BLOCKS, QKD), jnp.float32),
    )(h, W_group)

    out, mask, nn = pl.pallas_call(
        _attn_kernel,
        grid=(B // BT,),
        in_specs=[
            pl.BlockSpec((BT, S, INPUT_SIZE), lambda i: (i, 0, 0)),
            pl.BlockSpec((BT, NUM_BLOCKS, QKD), lambda i: (i, 0, 0)),
            pl.BlockSpec((INPUT_SIZE, QKD), lambda i: (0, 0)),
            pl.BlockSpec((INPUT_SIZE, VDIM), lambda i: (0, 0)),
        ],
        out_specs=[
            pl.BlockSpec((BT, NUM_BLOCKS, VDIM), lambda i: (i, 0, 0)),
            pl.BlockSpec((BT, NUM_BLOCKS), lambda i: (i, 0)),
            pl.BlockSpec((BT, NUM_BLOCKS), lambda i: (i, 0)),
        ],
        out_shape=[
            jax.ShapeDtypeStruct((B, NUM_BLOCKS, VDIM), jnp.float32),
            jax.ShapeDtypeStruct((B, NUM_BLOCKS), jnp.float32),
            jax.ShapeDtypeStruct((B, NUM_BLOCKS), jnp.float32),
        ],
    )(x, q, W_key, wv_eff)

    return out, mask, jax.lax.stop_gradient(nn)
